# baseline (speedup 1.0000x reference)
import functools

import jax
import jax.numpy as jnp
from jax import lax
from jax.experimental import pallas as pl
from jax.experimental.pallas import tpu as pltpu
from jax.experimental.pallas import tpu_sc as plsc

VOCAB = 1000000
DIM = 64
NUM_CORES = 2
NUM_SUBCORES = 16
NW = NUM_CORES * NUM_SUBCORES
GRP = 128


RB = 160
NRB = VOCAB // RB
RB_PER_W = -(-NRB // NW)


def _make_repack():
    mesh = plsc.VectorSubcoreMesh(core_axis_name="c", subcore_axis_name="s")

    @functools.partial(
        pl.kernel,
        mesh=mesh,
        out_type=jax.ShapeDtypeStruct((VOCAB // 2, 2 * DIM), jnp.float32),
        scratch_types=[
            [pltpu.VMEM((RB, DIM), jnp.float32) for _ in range(2)],
            [pltpu.VMEM((RB // 2, 2 * DIM), jnp.float32) for _ in range(2)],
            [pltpu.SemaphoreType.DMA for _ in range(2)],
            [pltpu.SemaphoreType.DMA for _ in range(2)],
        ],
        compiler_params=pltpu.CompilerParams(
            use_tc_tiling_on_sc=True, needs_layout_passes=False
        ),
    )
    def krep(tin, tout, rbuf, pbuf, rsem, wsem):
        wid = lax.axis_index("s") * NUM_CORES + lax.axis_index("c")
        lo = wid * RB_PER_W

        def fire_read(g, b):
            pltpu.async_copy(tin.at[pl.ds(g * RB, RB)], rbuf[b], rsem[b])

        def wait_read(b):
            pltpu.make_async_copy(
                tin.at[pl.ds(0, RB)], rbuf[b], rsem[b]
            ).wait()

        def drain_write(b):
            pltpu.make_async_copy(
                pbuf[b], tout.at[pl.ds(0, RB // 2)], wsem[b]
            ).wait()

        def repack(b):
            @plsc.parallel_loop(0, RB // 2, unroll=4)
            def _(r):
                for half in (0, 1):
                    for c in range(DIM // 16):
                        pbuf[b][r, pl.ds(half * DIM + c * 16, 16)] = rbuf[b][
                            2 * r + half, pl.ds(c * 16, 16)
                        ]

        def fire_write(g, b):
            pltpu.async_copy(
                pbuf[b], tout.at[pl.ds(g * (RB // 2), RB // 2)], wsem[b]
            )

        @pl.when(lo < NRB)
        def _():
            fire_read(lo, 0)

        def body2(i2, carry):
            for b in (0, 1):
                i = i2 * 2 + b
                g = lo + i

                @pl.when(jnp.logical_and(i + 1 < RB_PER_W, g + 1 < NRB))
                def _():
                    fire_read(g + 1, 1 - b)

                @pl.when(jnp.logical_and(i < RB_PER_W, g < NRB))
                def _():
                    wait_read(b)

                    @pl.when(i >= 2)
                    def _():
                        drain_write(b)

                    repack(b)
                    fire_write(g, b)

            return carry

        lax.fori_loop(0, (RB_PER_W + 2) // 2, body2, 0)
        drain_write(0)
        drain_write(1)

    return krep


def _make_gather(n_tokens: int):
    per_w = n_tokens // NW
    n_grp = per_w // GRP
    mesh = plsc.VectorSubcoreMesh(core_axis_name="c", subcore_axis_name="s")

    @functools.partial(
        pl.kernel,
        mesh=mesh,
        out_type=jax.ShapeDtypeStruct((n_tokens, DIM), jnp.float32),
        scratch_types=[
            pltpu.VMEM((per_w,), jnp.int32),
            [pltpu.VMEM((GRP,), jnp.int32) for _ in range(2)],
            [pltpu.VMEM((GRP,), jnp.int32) for _ in range(2)],
            [pltpu.VMEM((GRP, 2 * DIM), jnp.float32) for _ in range(2)],
            [pltpu.VMEM((GRP, DIM), jnp.float32) for _ in range(2)],
            [pltpu.SemaphoreType.DMA for _ in range(2)],
            [pltpu.SemaphoreType.DMA for _ in range(2)],
        ],
        compiler_params=pltpu.CompilerParams(
            use_tc_tiling_on_sc=True, needs_layout_passes=False
        ),
    )
    def kgat(xf, tab, out, ixbuf, qbuf, hbuf, stag, tbuf, gsem, wsem):
        wid = lax.axis_index("s") * NUM_CORES + lax.axis_index("c")
        base = wid * per_w
        iota = lax.iota(jnp.int32, 16)

        pltpu.sync_copy(xf.at[pl.ds(base, per_w)], ixbuf)

        def prep(g, b):
            for j in range(8):
                ids = ixbuf[pl.ds(g * GRP + j * 16, 16)]
                qbuf[b][pl.ds(j * 16, 16)] = lax.shift_right_logical(ids, 1)
                hbuf[b][pl.ds(j * 16, 16)] = lax.shift_left(
                    lax.bitwise_and(ids, 1), 6
                )

        def fire_gather(b):
            pltpu.async_copy(tab.at[qbuf[b]], stag[b], gsem[b])

        def wait_gather(b):
            pltpu.make_async_copy(tab.at[qbuf[b]], stag[b], gsem[b]).wait()

        def drain_write(b):
            pltpu.make_async_copy(
                tbuf[b], out.at[pl.ds(0, GRP)], wsem[b]
            ).wait()

        def extract(b):
            @plsc.parallel_loop(0, GRP, unroll=4)
            def _(k):
                ksplat = jnp.broadcast_to(k, (16,))
                hv = plsc.load_gather(hbuf[b], [ksplat])
                for c in range(DIM // 16):
                    idx = hv + (c * 16 + iota)
                    vals = plsc.load_gather(stag[b], [ksplat, idx])
                    tbuf[b][k, pl.ds(c * 16, 16)] = vals

        def fire_write(g, b):
            pltpu.async_copy(
                tbuf[b], out.at[pl.ds(base + g * GRP, GRP)], wsem[b]
            )

        prep(0, 0)
        fire_gather(0)

        def body2(i2, carry):
            for b in (0, 1):
                g = i2 * 2 + b

                @pl.when(g + 1 < n_grp)
                def _():
                    prep(g + 1, 1 - b)
                    fire_gather(1 - b)

                wait_gather(b)

                @pl.when(g >= 2)
                def _():
                    drain_write(b)

                extract(b)
                fire_write(g, b)

            return carry

        lax.fori_loop(0, n_grp // 2, body2, 0)
        drain_write(0)
        drain_write(1)

    return kgat


def kernel(x, table):
    batch, seq = x.shape
    n_tokens = batch * seq
    tab2 = _make_repack()(table)
    xf = x.reshape(n_tokens).astype(jnp.int32)
    out = _make_gather(n_tokens)(xf, tab2)
    return out.reshape(batch, seq, DIM)


# speedup vs baseline: 1.0101x; 1.0101x over previous
import functools

import jax
import jax.numpy as jnp
from jax import lax
from jax.experimental import pallas as pl
from jax.experimental.pallas import tpu as pltpu
from jax.experimental.pallas import tpu_sc as plsc

VOCAB = 1000000
DIM = 64
NUM_CORES = 2
NUM_SUBCORES = 16
NW = NUM_CORES * NUM_SUBCORES
GRP = 128


def _make_gather(n_tokens: int):
    per_w = n_tokens // NW
    n_grp = per_w // GRP
    mesh = plsc.VectorSubcoreMesh(core_axis_name="c", subcore_axis_name="s")

    @functools.partial(
        pl.kernel,
        mesh=mesh,
        out_type=jax.ShapeDtypeStruct((n_tokens, DIM), jnp.float32),
        scratch_types=[
            pltpu.VMEM((per_w,), jnp.int32),
            [pltpu.VMEM((GRP,), jnp.int32) for _ in range(2)],
            [pltpu.VMEM((GRP,), jnp.int32) for _ in range(2)],
            [pltpu.VMEM((GRP, 2 * DIM), jnp.float32) for _ in range(2)],
            [pltpu.VMEM((GRP, DIM), jnp.float32) for _ in range(2)],
            [pltpu.SemaphoreType.DMA for _ in range(2)],
            [pltpu.SemaphoreType.DMA for _ in range(2)],
        ],
        compiler_params=pltpu.CompilerParams(
            use_tc_tiling_on_sc=True, needs_layout_passes=False
        ),
    )
    def kgat(xf, tab, out, ixbuf, qbuf, hbuf, stag, tbuf, gsem, wsem):
        wid = lax.axis_index("s") * NUM_CORES + lax.axis_index("c")
        base = wid * per_w
        iota = lax.iota(jnp.int32, 16)

        pltpu.sync_copy(xf.at[pl.ds(base, per_w)], ixbuf)

        def prep(g, b):
            for j in range(8):
                ids = ixbuf[pl.ds(g * GRP + j * 16, 16)]
                qbuf[b][pl.ds(j * 16, 16)] = lax.shift_right_logical(ids, 1)
                hbuf[b][pl.ds(j * 16, 16)] = lax.shift_left(
                    lax.bitwise_and(ids, 1), 6
                )

        def fire_gather(b):
            pltpu.async_copy(tab.at[qbuf[b]], stag[b], gsem[b])

        def wait_gather(b):
            pltpu.make_async_copy(tab.at[qbuf[b]], stag[b], gsem[b]).wait()

        def drain_write(b):
            pltpu.make_async_copy(
                tbuf[b], out.at[pl.ds(0, GRP)], wsem[b]
            ).wait()

        def extract(b):
            @plsc.parallel_loop(0, GRP, unroll=4)
            def _(k):
                ksplat = jnp.broadcast_to(k, (16,))
                hv = plsc.load_gather(hbuf[b], [ksplat])
                for c in range(DIM // 16):
                    idx = hv + (c * 16 + iota)
                    vals = plsc.load_gather(stag[b], [ksplat, idx])
                    tbuf[b][k, pl.ds(c * 16, 16)] = vals

        def fire_write(g, b):
            pltpu.async_copy(
                tbuf[b], out.at[pl.ds(base + g * GRP, GRP)], wsem[b]
            )

        prep(0, 0)
        fire_gather(0)

        def body2(i2, carry):
            for b in (0, 1):
                g = i2 * 2 + b

                @pl.when(g + 1 < n_grp)
                def _():
                    prep(g + 1, 1 - b)
                    fire_gather(1 - b)

                wait_gather(b)

                @pl.when(g >= 2)
                def _():
                    drain_write(b)

                extract(b)
                fire_write(g, b)

            return carry

        lax.fori_loop(0, n_grp // 2, body2, 0)
        drain_write(0)
        drain_write(1)

    return kgat


def kernel(x, table):
    batch, seq = x.shape
    n_tokens = batch * seq
    tab2 = table.reshape(VOCAB // 2, 2 * DIM)
    xf = x.reshape(n_tokens).astype(jnp.int32)
    out = _make_gather(n_tokens)(xf, tab2)
    return out.reshape(batch, seq, DIM)
